# Initial kernel scaffold; baseline (speedup 1.0000x reference)
import jax
import jax.numpy as jnp
from jax.experimental import pallas as pl


def kernel(dst_input, src_attr, dst_index, W_lin, b_lin, w_dtp, W_proj, b_proj):
    raise NotImplementedError("write your pallas kernel here")


# speedup vs baseline: 38.6301x; 38.6301x over previous
import functools
import math

import jax
import jax.numpy as jnp
from jax import lax
from jax.experimental import pallas as pl
from jax.experimental.pallas import tpu as pltpu
from jax.experimental.pallas import tpu_sc as plsc

L = 16


def _sc_segment_sums(dst_index, src_attr_flat, n_pad, num_workers):
    e = dst_index.shape[0]
    e_per_w = e // num_workers
    mesh = plsc.VectorSubcoreMesh(core_axis_name="c", subcore_axis_name="s")

    @functools.partial(
        pl.kernel,
        out_type=(
            jax.ShapeDtypeStruct((num_workers, n_pad), jnp.float32),
            jax.ShapeDtypeStruct((num_workers, n_pad), jnp.float32),
        ),
        mesh=mesh,
        compiler_params=pltpu.CompilerParams(needs_layout_passes=False),
        scratch_types=(
            pltpu.VMEM((e_per_w,), jnp.int32),
            pltpu.VMEM((e_per_w,), jnp.float32),
            pltpu.VMEM((n_pad,), jnp.float32),
            pltpu.VMEM((n_pad,), jnp.float32),
        ),
    )
    def seg_kernel(idx_hbm, attr_hbm, s_out, c_out, idx_v, attr_v, s_acc, c_acc):
        wid = lax.axis_index("s") * 2 + lax.axis_index("c")
        base = wid * e_per_w
        pltpu.sync_copy(idx_hbm.at[pl.ds(base, e_per_w)], idx_v)
        pltpu.sync_copy(attr_hbm.at[pl.ds(base, e_per_w)], attr_v)

        zeros = jnp.zeros((L,), jnp.float32)

        def zero_body(i, carry):
            s_acc[pl.ds(i * L, L)] = zeros
            c_acc[pl.ds(i * L, L)] = zeros
            return carry

        lax.fori_loop(0, n_pad // L, zero_body, 0)

        ones = jnp.ones((L,), jnp.float32)

        def body(i, carry):
            sl = pl.ds(i * L, L)
            idx = idx_v[sl]
            val = attr_v[sl]
            plsc.addupdate_scatter(s_acc, [idx], val)
            plsc.addupdate_scatter(c_acc, [idx], ones)
            return carry

        lax.fori_loop(0, e_per_w // L, body, 0)

        pltpu.sync_copy(s_acc, s_out.at[wid])
        pltpu.sync_copy(c_acc, c_out.at[wid])

    return seg_kernel(dst_index, src_attr_flat)


def _tc_combine(ps, pc, wlin_row, blin_row, wdtp_row, W_proj, bproj_row, scale):
    nw, n_pad = ps.shape
    c = W_proj.shape[0]

    def body(ps_ref, pc_ref, wlin_ref, blin_ref, wdtp_ref, wproj_ref,
             bproj_ref, out_ref):
        dvec = (wlin_ref[...] + blin_ref[...]) * wdtp_ref[...]
        u = lax.dot_general(dvec, wproj_ref[...], (((1,), (1,)), ((), ())),
                            preferred_element_type=jnp.float32)
        u_rows = jnp.broadcast_to(u * scale, (nw, c))
        b_rows = jnp.broadcast_to(bproj_ref[...] * scale, (nw, c))
        out = lax.dot_general(ps_ref[...], u_rows, (((0,), (0,)), ((), ())),
                              preferred_element_type=jnp.float32)
        out += lax.dot_general(pc_ref[...], b_rows, (((0,), (0,)), ((), ())),
                               preferred_element_type=jnp.float32)
        out_ref[...] = out

    return pl.pallas_call(
        body,
        out_shape=jax.ShapeDtypeStruct((n_pad, c), jnp.float32),
    )(ps, pc, wlin_row, blin_row, wdtp_row, W_proj, bproj_row)


def kernel(dst_input, src_attr, dst_index, W_lin, b_lin, w_dtp, W_proj, b_proj):
    n, c = dst_input.shape
    e = dst_index.shape[0]
    num_workers = 32
    n_pad = ((n + 127) // 128) * 128
    scale = 1.0 / math.sqrt(32.0)

    ps, pc = _sc_segment_sums(dst_index, src_attr.reshape(e), n_pad, num_workers)
    out = _tc_combine(ps, pc, W_lin.reshape(1, c), b_lin.reshape(1, c),
                      w_dtp.reshape(1, c), W_proj, b_proj.reshape(1, c), scale)
    return out[:n]
